# Initial kernel scaffold; baseline (speedup 1.0000x reference)
import functools

import jax
import jax.numpy as jnp
from jax import lax
from jax.experimental import pallas as pl
from jax.experimental.pallas import tpu as pltpu
from jax.experimental.pallas import tpu_sc as plsc

NUM_WORKERS = 32
CHUNK = 128


@functools.partial(jax.jit, static_argnames=("n_chunks", "embed"))
def _gather_call(x_resh, w_embed, n_chunks, embed):
    b_per_w = n_chunks * CHUNK
    total = NUM_WORKERS * b_per_w

    mesh = plsc.VectorSubcoreMesh(core_axis_name="c", subcore_axis_name="s")

    @functools.partial(
        pl.kernel,
        out_type=jax.ShapeDtypeStruct((total, embed), jnp.float32),
        mesh=mesh,
        scratch_types=[
            pltpu.VMEM((n_chunks, CHUNK), jnp.int32),
            pltpu.VMEM((CHUNK, embed), jnp.float32),
            pltpu.SemaphoreType.DMA,
        ],
    )
    def gather_kernel(x_hbm, table_hbm, out_hbm, idx_v, buf, sem):
        wid = lax.axis_index("s") * 2 + lax.axis_index("c")
        base = wid * b_per_w
        pltpu.sync_copy(x_hbm.at[wid], idx_v)

        def body(c, carry):
            pltpu.async_copy(table_hbm.at[idx_v.at[c]], buf, sem).wait()
            pltpu.sync_copy(buf, out_hbm.at[pl.ds(base + c * CHUNK, CHUNK)])
            return carry

        lax.fori_loop(0, n_chunks, body, 0)

    return gather_kernel(x_resh, w_embed)


def kernel(x, W_embed):
    n, t = x.shape
    _, embed = W_embed.shape
    b = n * t
    assert b % (NUM_WORKERS * CHUNK) == 0
    n_chunks = b // (NUM_WORKERS * CHUNK)
    x_resh = x.reshape(NUM_WORKERS, n_chunks, CHUNK).astype(jnp.int32)
    out = _gather_call(x_resh, W_embed, n_chunks, embed)
    return out.reshape(n, t, embed)


# speedup vs baseline: 4.0967x; 4.0967x over previous
import functools

import jax
import jax.numpy as jnp
from jax import lax
from jax.experimental import pallas as pl
from jax.experimental.pallas import tpu as pltpu
from jax.experimental.pallas import tpu_sc as plsc

NUM_WORKERS = 32
CHUNK = 128


@functools.partial(jax.jit, static_argnames=("n_chunks", "embed"))
def _gather_call(x_resh, w_embed, n_chunks, embed):
    b_per_w = n_chunks * CHUNK
    total = NUM_WORKERS * b_per_w

    mesh = plsc.VectorSubcoreMesh(core_axis_name="c", subcore_axis_name="s")

    @functools.partial(
        pl.kernel,
        out_type=jax.ShapeDtypeStruct((total, embed), jnp.float32),
        mesh=mesh,
        scratch_types=[
            pltpu.VMEM((n_chunks, CHUNK), jnp.int32),
            pltpu.VMEM((CHUNK, embed), jnp.float32),
            pltpu.SemaphoreType.DMA,
        ],
        compiler_params=pltpu.CompilerParams(use_tc_tiling_on_sc=False),
    )
    def gather_kernel(x_hbm, table_hbm, out_hbm, idx_v, buf, sem):
        wid = lax.axis_index("s") * 2 + lax.axis_index("c")
        base = wid * b_per_w
        pltpu.sync_copy(x_hbm.at[wid], idx_v)

        def body(c, carry):
            pltpu.async_copy(table_hbm.at[idx_v.at[c]], buf, sem).wait()
            pltpu.sync_copy(buf, out_hbm.at[pl.ds(base + c * CHUNK, CHUNK)])
            return carry

        lax.fori_loop(0, n_chunks, body, 0)

    return gather_kernel(x_resh, w_embed)


def kernel(x, W_embed):
    n, t = x.shape
    _, embed = W_embed.shape
    b = n * t
    assert b % (NUM_WORKERS * CHUNK) == 0
    n_chunks = b // (NUM_WORKERS * CHUNK)
    x_resh = x.reshape(NUM_WORKERS, n_chunks, CHUNK).astype(jnp.int32)
    out = _gather_call(x_resh, W_embed, n_chunks, embed)
    return out.reshape(n, t, embed)


# speedup vs baseline: 4.5605x; 1.1132x over previous
import functools

import jax
import jax.numpy as jnp
from jax import lax
from jax.experimental import pallas as pl
from jax.experimental.pallas import tpu as pltpu
from jax.experimental.pallas import tpu_sc as plsc

NUM_WORKERS = 32
CHUNK = 128


@functools.partial(jax.jit, static_argnames=("n_chunks", "embed"))
def _gather_call(x_resh, w_embed, n_chunks, embed):
    b_per_w = n_chunks * CHUNK
    total = NUM_WORKERS * b_per_w

    mesh = plsc.VectorSubcoreMesh(core_axis_name="c", subcore_axis_name="s")

    @functools.partial(
        pl.kernel,
        out_type=jax.ShapeDtypeStruct((total, embed), jnp.float32),
        mesh=mesh,
        scratch_types=[
            pltpu.VMEM((n_chunks, CHUNK), jnp.int32),
            pltpu.VMEM((CHUNK, embed), jnp.float32),
            pltpu.VMEM((CHUNK, embed), jnp.float32),
            pltpu.SemaphoreType.DMA,
            pltpu.SemaphoreType.DMA,
        ],
        compiler_params=pltpu.CompilerParams(use_tc_tiling_on_sc=False),
    )
    def gather_kernel(x_hbm, table_hbm, out_hbm, idx_v, buf0, buf1, sem0, sem1):
        wid = lax.axis_index("s") * 2 + lax.axis_index("c")
        base = wid * b_per_w
        pltpu.sync_copy(x_hbm.at[wid], idx_v)

        def start_gather(c, buf, sem):
            pltpu.make_async_copy(table_hbm.at[idx_v.at[c]], buf, sem).start()

        def wait_and_write(c, buf, sem):
            pltpu.make_async_copy(table_hbm.at[idx_v.at[c]], buf, sem).wait()
            pltpu.sync_copy(buf, out_hbm.at[pl.ds(base + c * CHUNK, CHUNK)])

        start_gather(0, buf0, sem0)

        def body(i, carry):
            c0 = 2 * i
            start_gather(c0 + 1, buf1, sem1)
            wait_and_write(c0, buf0, sem0)
            start_gather(c0 + 2, buf0, sem0)
            wait_and_write(c0 + 1, buf1, sem1)
            return carry

        lax.fori_loop(0, n_chunks // 2 - 1, body, 0)
        last = n_chunks - 2
        start_gather(last + 1, buf1, sem1)
        wait_and_write(last, buf0, sem0)
        wait_and_write(last + 1, buf1, sem1)

    return gather_kernel(x_resh, w_embed)


def kernel(x, W_embed):
    n, t = x.shape
    _, embed = W_embed.shape
    b = n * t
    assert b % (NUM_WORKERS * CHUNK) == 0
    n_chunks = b // (NUM_WORKERS * CHUNK)
    x_resh = x.reshape(NUM_WORKERS, n_chunks, CHUNK).astype(jnp.int32)
    out = _gather_call(x_resh, W_embed, n_chunks, embed)
    return out.reshape(n, t, embed)


# speedup vs baseline: 4.6808x; 1.0264x over previous
import functools

import jax
import jax.numpy as jnp
from jax import lax
from jax.experimental import pallas as pl
from jax.experimental.pallas import tpu as pltpu
from jax.experimental.pallas import tpu_sc as plsc

NUM_WORKERS = 32
CHUNK = 128
RING = 5


@functools.partial(jax.jit, static_argnames=("n_chunks", "embed"))
def _gather_call(x_resh, w_embed, n_chunks, embed):
    b_per_w = n_chunks * CHUNK
    total = NUM_WORKERS * b_per_w

    mesh = plsc.VectorSubcoreMesh(core_axis_name="c", subcore_axis_name="s")

    @functools.partial(
        pl.kernel,
        out_type=jax.ShapeDtypeStruct((total, embed), jnp.float32),
        mesh=mesh,
        scratch_types=[
            pltpu.VMEM((n_chunks, CHUNK), jnp.int32),
        ]
        + [pltpu.VMEM((CHUNK, embed), jnp.float32) for _ in range(RING)]
        + [pltpu.SemaphoreType.DMA for _ in range(RING)],
        compiler_params=pltpu.CompilerParams(use_tc_tiling_on_sc=False),
    )
    def gather_kernel(x_hbm, table_hbm, out_hbm, idx_v, *scratch):
        bufs = scratch[:RING]
        sems = scratch[RING:]
        wid = lax.axis_index("s") * 2 + lax.axis_index("c")
        base = wid * b_per_w
        pltpu.sync_copy(x_hbm.at[wid], idx_v)

        def start_gather(c, buf, sem):
            pltpu.make_async_copy(table_hbm.at[idx_v.at[c]], buf, sem).start()

        def wait_and_write(c, buf, sem):
            pltpu.make_async_copy(table_hbm.at[idx_v.at[c]], buf, sem).wait()
            pltpu.sync_copy(buf, out_hbm.at[pl.ds(base + c * CHUNK, CHUNK)])

        for b in range(RING):
            start_gather(b, bufs[b], sems[b])

        def body(i, carry):
            c0 = RING * i
            for b in range(RING):
                wait_and_write(c0 + b, bufs[b], sems[b])
                start_gather(c0 + b + RING, bufs[b], sems[b])
            return carry

        lax.fori_loop(0, n_chunks // RING - 1, body, 0)
        c0 = n_chunks - RING
        for b in range(RING):
            wait_and_write(c0 + b, bufs[b], sems[b])

    return gather_kernel(x_resh, w_embed)


def kernel(x, W_embed):
    n, t = x.shape
    _, embed = W_embed.shape
    b = n * t
    assert b % (NUM_WORKERS * CHUNK) == 0
    n_chunks = b // (NUM_WORKERS * CHUNK)
    x_resh = x.reshape(NUM_WORKERS, n_chunks, CHUNK).astype(jnp.int32)
    out = _gather_call(x_resh, W_embed, n_chunks, embed)
    return out.reshape(n, t, embed)
